# Initial kernel scaffold; baseline (speedup 1.0000x reference)
import jax
import jax.numpy as jnp
from jax.experimental import pallas as pl


def kernel(x, adj, W1, b1, W2, att_src, att_dst, bias):
    raise NotImplementedError("write your pallas kernel here")


# speedup vs baseline: 12165.9639x; 12165.9639x over previous
import functools

import jax
import jax.numpy as jnp
from jax.experimental import pallas as pl
from jax.experimental.pallas import tpu as pltpu

N = 4096
D_OUT = 8


def _proj_kernel(x_ref, w1_ref, b1_ref, w2_ref, asrc_ref, adst_ref,
                 g_ref, a_s_ref, a_d_ref):
    x = x_ref[...]
    h = jax.lax.dot_general(x, w1_ref[...], (((1,), (1,)), ((), ())),
                            preferred_element_type=jnp.float32)
    h = h + b1_ref[...]
    h = jnp.where(h >= 0, h, 0.01 * h)
    g = jax.lax.dot_general(h, w2_ref[...], (((1,), (1,)), ((), ())),
                            preferred_element_type=jnp.float32)
    g_ref[...] = g
    a_s_ref[...] = jax.lax.dot_general(g, asrc_ref[...], (((1,), (0,)), ((), ())),
                                       preferred_element_type=jnp.float32)
    a_d_ref[...] = jax.lax.dot_general(g, adst_ref[...], (((1,), (0,)), ((), ())),
                                       preferred_element_type=jnp.float32)


def _attn_kernel(adj_ref, g_ref, a_s_ref, a_d_ref, bias_ref, out_ref,
                 m_ref, s_ref, acc_ref, *, ib, jb, ni):
    j = pl.program_id(0)
    i = pl.program_id(1)

    @pl.when(i == 0)
    def _init():
        m_ref[...] = jnp.full_like(m_ref, -3.4e38)
        s_ref[...] = jnp.zeros_like(s_ref)
        acc_ref[...] = jnp.zeros_like(acc_ref)

    a = adj_ref[...]
    rows = i * ib + jax.lax.broadcasted_iota(jnp.int32, (ib, jb), 0)
    cols = j * jb + jax.lax.broadcasted_iota(jnp.int32, (ib, jb), 1)
    mask = (a != 0) | (rows == cols)

    z = a_s_ref[...] + a_d_ref[...]
    e = jnp.where(z >= 0, z, 0.2 * z)

    e_m = jnp.where(mask, e, -3.4e38)
    bm = jnp.max(e_m, axis=0, keepdims=True)
    m_new = jnp.maximum(m_ref[...], bm)
    corr = jnp.exp(m_ref[...] - m_new)
    p = jnp.where(mask, jnp.exp(e - m_new), 0.0)

    m_ref[...] = m_new
    s_ref[...] = s_ref[...] * corr + jnp.sum(p, axis=0, keepdims=True)
    acc_ref[...] = acc_ref[...] * corr + jax.lax.dot_general(
        g_ref[...], p, (((0,), (0,)), ((), ())),
        preferred_element_type=jnp.float32)

    @pl.when(i == ni - 1)
    def _fini():
        out_ref[...] = acc_ref[...] / (s_ref[...] + 1e-16) + bias_ref[...]


@jax.jit
def kernel(x, adj, W1, b1, W2, att_src, att_dst, bias):
    n = x.shape[0]

    g, a_s, a_d = pl.pallas_call(
        _proj_kernel,
        out_shape=(
            jax.ShapeDtypeStruct((n, D_OUT), jnp.float32),
            jax.ShapeDtypeStruct((n, 1), jnp.float32),
            jax.ShapeDtypeStruct((n, 1), jnp.float32),
        ),
    )(x, W1, b1.reshape(1, -1), W2,
      att_src.reshape(-1, 1), att_dst.reshape(-1, 1))

    ib, jb = 1024, 512
    ni = n // ib
    nj = n // jb

    out_t = pl.pallas_call(
        functools.partial(_attn_kernel, ib=ib, jb=jb, ni=ni),
        grid=(nj, ni),
        in_specs=[
            pl.BlockSpec((ib, jb), lambda j, i: (i, j)),
            pl.BlockSpec((ib, D_OUT), lambda j, i: (i, 0)),
            pl.BlockSpec((ib, 1), lambda j, i: (i, 0)),
            pl.BlockSpec((1, jb), lambda j, i: (0, j)),
            pl.BlockSpec((D_OUT, 1), lambda j, i: (0, 0)),
        ],
        out_specs=pl.BlockSpec((D_OUT, jb), lambda j, i: (0, j)),
        out_shape=jax.ShapeDtypeStruct((D_OUT, n), jnp.float32),
        scratch_shapes=[
            pltpu.VMEM((1, jb), jnp.float32),
            pltpu.VMEM((1, jb), jnp.float32),
            pltpu.VMEM((D_OUT, jb), jnp.float32),
        ],
    )(adj, g, a_s, a_d.reshape(1, -1), bias.reshape(-1, 1))

    return out_t.T
